# baseline (speedup 1.0000x reference)
import functools

import jax
import jax.numpy as jnp
from jax import lax
from jax.experimental import pallas as pl
from jax.experimental.pallas import tpu as pltpu
from jax.experimental.pallas import tpu_sc as plsc

_N = 50000
_D = 128
_NUM_PATH = 3
_PATH_LEN = 4
_K = _NUM_PATH * _PATH_LEN
_NW = 32
_RPW = 1568
_N_PAD = _NW * _RPW
_C = 32
_NCH = _RPW // _C
_NGB = _C * _K // 128
_CI = _C * _K
_NP2 = _N_PAD + _C

_mesh = plsc.VectorSubcoreMesh(core_axis_name="c", subcore_axis_name="s")


@functools.partial(
    pl.kernel,
    mesh=_mesh,
    out_type=jax.ShapeDtypeStruct((_N_PAD, _D), jnp.float32),
    scratch_types=[
        pltpu.VMEM((2 * _CI,), jnp.int32),
        pltpu.VMEM((2 * _CI, _D), jnp.float32),
        pltpu.VMEM((2 * _C, _D), jnp.float32),
        pltpu.VMEM((_PATH_LEN, _D), jnp.float32),
        pltpu.SemaphoreType.DMA,
        pltpu.SemaphoreType.DMA,
        pltpu.SemaphoreType.DMA,
    ],
)
def _sc_gather_acc(feats_hbm, idx_hbm, pw_hbm, out_hbm,
                   idx_v, rows_v, out_v, pw_v, si, sg, so):
    wid = lax.axis_index("s") * 2 + lax.axis_index("c")
    pltpu.sync_copy(pw_hbm, pw_v)

    def idx_copies(ch, par):
        row0 = wid * _RPW + ch * _C
        return [
            pltpu.make_async_copy(
                idx_hbm.at[pl.ds(i * _NP2 * _PATH_LEN + row0 * _PATH_LEN, 128)],
                idx_v.at[pl.ds(par * _CI + i * 128, 128)],
                si)
            for i in range(_NUM_PATH)
        ]

    def gather_copies(par):
        return [
            pltpu.make_async_copy(
                feats_hbm.at[idx_v.at[pl.ds(par * _CI + g * 128, 128)]],
                rows_v.at[pl.ds(par * _CI + g * 128, 128)],
                sg)
            for g in range(_NGB)
        ]

    def out_copy(ch, par):
        row0 = wid * _RPW + ch * _C
        return pltpu.make_async_copy(
            out_v.at[pl.ds(par * _C, _C)],
            out_hbm.at[pl.ds(row0, _C)],
            so)

    def compute(par):
        base_r = par * _CI
        base_o = par * _C
        for v in range(_D // 16):
            sl = pl.ds(v * 16, 16)
            pws = tuple(pw_v[j, sl] for j in range(_PATH_LEN))

            def row_body(c, acc_carry, _sl=sl, _pws=pws):
                b0 = base_r + c * _PATH_LEN
                acc = rows_v[b0, _sl] * _pws[0]
                for k in range(1, _K):
                    i, j = divmod(k, _PATH_LEN)
                    acc = acc + rows_v[b0 + i * 128 + j, _sl] * _pws[j]
                out_v[base_o + c, _sl] = acc
                return acc_carry

            lax.fori_loop(0, _C, row_body, 0)

    for cp in idx_copies(0, 0):
        cp.start()
    for cp in idx_copies(1, 1):
        cp.start()
    for cp in idx_copies(0, 0):
        cp.wait()
    for cp in gather_copies(0):
        cp.start()
    out_copy(0, 0).start()
    out_copy(1, 1).start()

    def body(ch, carry):
        par = lax.rem(ch, 2)
        npar = 1 - par
        for cp in idx_copies(ch + 1, npar):
            cp.wait()
        for cp in gather_copies(npar):
            cp.start()
        for cp in gather_copies(par):
            cp.wait()
        for cp in idx_copies(ch + 2, par):
            cp.start()
        out_copy(ch, par).wait()
        compute(par)
        out_copy(ch, par).start()
        return carry

    lax.fori_loop(0, _NCH - 1, body, 0)
    lpar = (_NCH - 1) % 2
    for cp in gather_copies(lpar):
        cp.wait()
    out_copy(_NCH - 1, lpar).wait()
    compute(lpar)
    out_copy(_NCH - 1, lpar).start()
    out_copy(_NCH - 2, 1 - lpar).wait()
    out_copy(_NCH - 1, lpar).wait()


_BN = 2000


def _mm_body(x_ref, w_ref, o_ref):
    o_ref[...] = jnp.maximum(
        lax.dot_general(x_ref[...], w_ref[...],
                        (((1,), (1,)), ((), ())),
                        preferred_element_type=jnp.float32),
        0.0)


def _tc_mm_relu(x, w):
    return pl.pallas_call(
        _mm_body,
        grid=(_N // _BN,),
        in_specs=[
            pl.BlockSpec((_BN, _D), lambda i: (i, 0)),
            pl.BlockSpec((_D, _D), lambda i: (0, 0)),
        ],
        out_specs=pl.BlockSpec((_BN, _D), lambda i: (i, 0)),
        out_shape=jax.ShapeDtypeStruct((_N, _D), jnp.float32),
    )(x, w)


def kernel(feats, paths, init_feats, path_weight, fc_weight):
    del init_feats
    p32 = paths.astype(jnp.int32)
    idx_flat = jnp.pad(p32, ((0, 0), (0, _NP2 - _N), (0, 0))).reshape(-1)
    pw = path_weight[0] * (1.0 / _NUM_PATH)
    acc = _sc_gather_acc(feats, idx_flat, pw)
    return _tc_mm_relu(acc, fc_weight)


# speedup vs baseline: 1.2374x; 1.2374x over previous
import functools

import jax
import jax.numpy as jnp
from jax import lax
from jax.experimental import pallas as pl
from jax.experimental.pallas import tpu as pltpu
from jax.experimental.pallas import tpu_sc as plsc

_N = 50000
_D = 128
_NUM_PATH = 3
_PATH_LEN = 4
_K = _NUM_PATH * _PATH_LEN
_NW = 32
_RPW = 1568
_N_PAD = _NW * _RPW
_C = 32
_NCH = _RPW // _C
_NGB = _C * _K // 128
_CI = _C * _K

_mesh = plsc.VectorSubcoreMesh(core_axis_name="c", subcore_axis_name="s")


@functools.partial(
    pl.kernel,
    mesh=_mesh,
    out_type=jax.ShapeDtypeStruct((_N_PAD, _D), jnp.float32),
    scratch_types=[
        pltpu.VMEM((2 * _CI,), jnp.int32),
        pltpu.VMEM((2 * _CI, _D), jnp.float32),
        pltpu.VMEM((2 * _C, _D), jnp.float32),
        pltpu.VMEM((_PATH_LEN, _D), jnp.float32),
        pltpu.SemaphoreType.DMA,
        pltpu.SemaphoreType.DMA,
        pltpu.SemaphoreType.DMA,
    ],
)
def _sc_gather_acc(feats_hbm, idx_hbm, pw_hbm, out_hbm,
                   idx_v, rows_v, out_v, pw_v, si, sg, so):
    wid = lax.axis_index("c") * 16 + lax.axis_index("s")
    pltpu.sync_copy(pw_hbm, pw_v)

    def idx_copy(ch, par):
        row0 = wid * _RPW + ch * _C
        return pltpu.make_async_copy(
            idx_hbm.at[pl.ds(row0 * _K, _CI)],
            idx_v.at[pl.ds(par * _CI, _CI)],
            si)

    def gather_copies(par):
        return [
            pltpu.make_async_copy(
                feats_hbm.at[idx_v.at[pl.ds(par * _CI + g * 128, 128)]],
                rows_v.at[pl.ds(par * _CI + g * 128, 128)],
                sg)
            for g in range(_NGB)
        ]

    def out_copy(ch, par):
        row0 = wid * _RPW + ch * _C
        return pltpu.make_async_copy(
            out_v.at[pl.ds(par * _C, _C)],
            out_hbm.at[pl.ds(row0, _C)],
            so)

    def compute(par):
        base_r = par * _CI
        base_o = par * _C
        for v in range(_D // 16):
            sl = pl.ds(v * 16, 16)
            pws = tuple(pw_v[j, sl] for j in range(_PATH_LEN))

            def row_body(c, acc_carry, _sl=sl, _pws=pws):
                b0 = base_r + c * _K
                acc = rows_v[b0, _sl] * _pws[0]
                for k in range(1, _K):
                    acc = acc + rows_v[b0 + k, _sl] * _pws[k % _PATH_LEN]
                out_v[base_o + c, _sl] = acc
                return acc_carry

            lax.fori_loop(0, _C, row_body, 0)

    idx_copy(0, 0).start()
    idx_copy(1, 1).start()
    idx_copy(0, 0).wait()
    for cp in gather_copies(0):
        cp.start()
    out_copy(0, 0).start()
    out_copy(1, 1).start()

    def body(ch, carry):
        par = lax.rem(ch, 2)
        npar = 1 - par
        idx_copy(ch + 1, npar).wait()
        for cp in gather_copies(npar):
            cp.start()
        for cp in gather_copies(par):
            cp.wait()
        idx_copy(ch + 2, par).start()
        out_copy(ch, par).wait()
        compute(par)
        out_copy(ch, par).start()
        return carry

    lax.fori_loop(0, _NCH - 1, body, 0)
    lpar = (_NCH - 1) % 2
    for cp in gather_copies(lpar):
        cp.wait()
    out_copy(_NCH - 1, lpar).wait()
    compute(lpar)
    out_copy(_NCH - 1, lpar).start()
    out_copy(_NCH - 2, 1 - lpar).wait()
    out_copy(_NCH - 1, lpar).wait()


_BN = 2000


def _mm_body(x_ref, w_ref, o_ref):
    o_ref[...] = jnp.maximum(
        lax.dot_general(x_ref[...], w_ref[...],
                        (((1,), (1,)), ((), ())),
                        preferred_element_type=jnp.float32),
        0.0)


def _tc_mm_relu(x, w):
    return pl.pallas_call(
        _mm_body,
        grid=(_N // _BN,),
        in_specs=[
            pl.BlockSpec((_BN, _D), lambda i: (i, 0)),
            pl.BlockSpec((_D, _D), lambda i: (0, 0)),
        ],
        out_specs=pl.BlockSpec((_BN, _D), lambda i: (i, 0)),
        out_shape=jax.ShapeDtypeStruct((_N, _D), jnp.float32),
    )(x, w)


def kernel(feats, paths, init_feats, path_weight, fc_weight):
    del init_feats
    idx = jnp.transpose(paths.astype(jnp.int32), (1, 0, 2)).reshape(_N, _K)
    idx_flat = jnp.pad(idx, ((0, _N_PAD + _C - _N), (0, 0))).reshape(-1)
    pw = path_weight[0] * (1.0 / _NUM_PATH)
    acc = _sc_gather_acc(feats, idx_flat, pw)
    return _tc_mm_relu(acc, fc_weight)
